# baseline (speedup 1.0000x reference)
import functools

import jax
import jax.numpy as jnp
from jax import lax
from jax.experimental import pallas as pl
from jax.experimental.pallas import tpu as pltpu
from jax.experimental.pallas import tpu_sc as plsc

VOCAB = 1000000
D = 64
B = 4096 * 200
NC, NS = 2, 16
NW = NC * NS
B_PER_W = B // NW
K = 8
C = K * 128
STEPS = B_PER_W // C


def _mesh():
    return plsc.VectorSubcoreMesh(core_axis_name="c", subcore_axis_name="s")


@functools.partial(
    pl.kernel,
    mesh=_mesh(),
    out_type=jax.ShapeDtypeStruct((B, D), jnp.float32),
    scratch_types=[
        pltpu.VMEM((K, 128), jnp.int32),
        pltpu.VMEM((C, D), jnp.float32),
        pltpu.SemaphoreType.DMA,
    ],
    compiler_params=pltpu.CompilerParams(use_tc_tiling_on_sc=False),
)
def _gather_kernel(table_hbm, idx_hbm, out_hbm, idx_v, rows_v, sem):
    wid = lax.axis_index("s") * NC + lax.axis_index("c")

    def step(s, carry):
        row0 = wid * (B_PER_W // 128) + s * K
        base = wid * B_PER_W + s * C
        pltpu.sync_copy(idx_hbm.at[pl.ds(row0, K)], idx_v)
        copies = [
            pltpu.async_copy(
                table_hbm.at[idx_v.at[j]],
                rows_v.at[pl.ds(j * 128, 128)],
                sem,
            )
            for j in range(K)
        ]
        for cp in copies:
            cp.wait()
        pltpu.sync_copy(rows_v, out_hbm.at[pl.ds(base, C)])
        return carry

    lax.fori_loop(0, STEPS, step, 0)


def kernel(x, table):
    idx = x.reshape(B // 128, 128)
    out = _gather_kernel(table, idx)
    return out.reshape(x.shape[0], x.shape[1], D)


# speedup vs baseline: 1.0167x; 1.0167x over previous
import functools

import jax
import jax.numpy as jnp
from jax import lax
from jax.experimental import pallas as pl
from jax.experimental.pallas import tpu as pltpu
from jax.experimental.pallas import tpu_sc as plsc

VOCAB = 1000000
D = 64
B = 4096 * 200
NC, NS = 2, 16
NW = NC * NS
B_PER_W = B // NW
IROWS = B_PER_W // 128
K = 4
C = K * 128
STEPS = B_PER_W // C
G = STEPS // 2


def _mesh():
    return plsc.VectorSubcoreMesh(core_axis_name="c", subcore_axis_name="s")


@functools.partial(
    pl.kernel,
    mesh=_mesh(),
    out_type=jax.ShapeDtypeStruct((B, D), jnp.float32),
    scratch_types=[
        pltpu.VMEM((IROWS, 128), jnp.int32),
        pltpu.VMEM((C, D), jnp.float32),
        pltpu.VMEM((C, D), jnp.float32),
        pltpu.SemaphoreType.DMA,
        pltpu.SemaphoreType.DMA,
        pltpu.SemaphoreType.DMA,
    ],
    compiler_params=pltpu.CompilerParams(use_tc_tiling_on_sc=False),
)
def _gather_kernel(table_hbm, idx_hbm, out_hbm, idx_v, rows0, rows1,
                   sem_a, sem_b, sem_st):
    wid = lax.axis_index("s") * NC + lax.axis_index("c")
    obase = wid * B_PER_W

    pltpu.sync_copy(idx_hbm.at[pl.ds(wid * IROWS, IROWS)], idx_v)

    def fire_gather(chunk, rows_v, sem):
        return [
            pltpu.async_copy(
                table_hbm.at[idx_v.at[chunk * K + j]],
                rows_v.at[pl.ds(j * 128, 128)],
                sem,
            )
            for j in range(K)
        ]

    def drain(copies):
        for cp in copies:
            cp.wait()

    def store(chunk, rows_v):
        return pltpu.async_copy(
            rows_v, out_hbm.at[pl.ds(obase + chunk * C, C)], sem_st)

    drain(fire_gather(0, rows0, sem_a))
    st = store(0, rows0)

    def body(g, carry):
        c0 = 2 * g
        gb = fire_gather(c0 + 1, rows1, sem_b)
        st.wait()

        @pl.when(g < G - 1)
        def _():
            ga = fire_gather(c0 + 2, rows0, sem_a)
            drain(gb)
            store(c0 + 1, rows1).wait()
            drain(ga)
            st2 = store(c0 + 2, rows0)
            del st2

        @pl.when(g == G - 1)
        def _():
            drain(gb)
            store(c0 + 1, rows1).wait()

        return carry

    lax.fori_loop(0, G, body, 0)


def kernel(x, table):
    idx = x.reshape(B // 128, 128)
    out = _gather_kernel(table, idx)
    return out.reshape(x.shape[0], x.shape[1], D)
